# baseline (speedup 1.0000x reference)
import jax
import jax.numpy as jnp
from jax.experimental import pallas as pl
from jax.experimental.pallas import tpu as pltpu

_TM = 200
_NSLOT = 4


def _gcn_body(a_hbm, x_ref, w_ref, o_ref, buf, sem):
    n = x_ref.shape[0]
    nblk = n // _TM
    wt = w_ref[...].T

    def _copy(slot, blk):
        return pltpu.make_async_copy(
            a_hbm.at[pl.ds(blk * _TM, _TM), :], buf.at[slot], sem.at[slot]
        )

    for s in range(_NSLOT):
        _copy(s, s).start()

    def _step(blk, carry):
        slot = jax.lax.rem(blk, _NSLOT)
        _copy(slot, blk).wait()
        ax = jnp.dot(buf[slot], x_ref[...], preferred_element_type=jnp.float32)
        o_ref[pl.ds(blk * _TM, _TM), :] = jnp.dot(
            ax, wt, preferred_element_type=jnp.float32
        )

        @pl.when(blk + _NSLOT < nblk)
        def _():
            _copy(slot, blk + _NSLOT).start()

        return carry

    jax.lax.fori_loop(0, nblk, _step, 0)


def kernel(X, A_tilde, W):
    n, d_in = X.shape
    d_out = W.shape[0]
    return pl.pallas_call(
        _gcn_body,
        in_specs=[
            pl.BlockSpec(memory_space=pl.ANY),
            pl.BlockSpec(memory_space=pltpu.VMEM),
            pl.BlockSpec(memory_space=pltpu.VMEM),
        ],
        out_specs=pl.BlockSpec(memory_space=pltpu.VMEM),
        out_shape=jax.ShapeDtypeStruct((n, d_out), jnp.float32),
        scratch_shapes=[
            pltpu.VMEM((_NSLOT, _TM, n), jnp.float32),
            pltpu.SemaphoreType.DMA((_NSLOT,)),
        ],
    )(A_tilde, X, W)


# speedup vs baseline: 1.0322x; 1.0322x over previous
import jax
import jax.numpy as jnp
from jax.experimental import pallas as pl
from jax.experimental.pallas import tpu as pltpu

_TM = 400


def _gcn_block(a_ref, x_ref, w_ref, o_ref):
    ax = jnp.dot(a_ref[...], x_ref[...], preferred_element_type=jnp.float32)
    o_ref[...] = jnp.dot(ax, w_ref[...].T, preferred_element_type=jnp.float32)


def kernel(X, A_tilde, W):
    n, d_in = X.shape
    d_out = W.shape[0]
    return pl.pallas_call(
        _gcn_block,
        grid=(n // _TM,),
        in_specs=[
            pl.BlockSpec((_TM, n), lambda i: (i, 0)),
            pl.BlockSpec((n, d_in), lambda i: (0, 0)),
            pl.BlockSpec((d_out, d_in), lambda i: (0, 0)),
        ],
        out_specs=pl.BlockSpec((_TM, d_out), lambda i: (i, 0)),
        out_shape=jax.ShapeDtypeStruct((n, d_out), jnp.float32),
        compiler_params=pltpu.CompilerParams(dimension_semantics=("parallel",)),
    )(A_tilde, X, W)
